# Initial kernel scaffold; baseline (speedup 1.0000x reference)
import jax
import jax.numpy as jnp
from jax.experimental import pallas as pl


def kernel(x, domain_idx, W_enc, b_enc, rotors, W_router, b_router, W_exp, b_exp, W_inv, b_inv, Wq, mem_keys, mem_vals, W_head, b_head, W_out, b_out):
    raise NotImplementedError("write your pallas kernel here")


# speedup vs baseline: 1.7215x; 1.7215x over previous
import functools

import jax
import jax.numpy as jnp
from jax import lax
from jax.experimental import pallas as pl
from jax.experimental.pallas import tpu as pltpu

D = 1024
E = 8
K = 2
CD = 16
MKD = 32
M = 512
N_TOK = 2048

TBLK = 256


def _fused_body(x_ref, rotor_ref, W_enc_ref, b_enc_ref, W_router_ref,
                b_router_ref, W_exp_ref, b_exp_ref, W_inv_ref, b_inv_ref,
                Wq_ref, mem_keys_ref, mem_vals_ref, W_head_ref, b_head_ref,
                W_out_ref, b_out_ref,
                out_ref, rw_ref, tinv_ref):
    x = x_ref[...]
    h = jax.nn.gelu(jnp.dot(x, W_enc_ref[...],
                            preferred_element_type=jnp.float32)
                    + b_enc_ref[...][None, :])
    h = h * rotor_ref[...][None, :]

    logits = (jnp.dot(h, W_router_ref[...], preferred_element_type=jnp.float32)
              + b_router_ref[...][None, :])
    z = logits - jnp.max(logits, axis=1, keepdims=True)
    ez = jnp.exp(z)
    probs = ez / jnp.sum(ez, axis=1, keepdims=True)

    iota8 = lax.broadcasted_iota(jnp.int32, (TBLK, E), 1)
    m1 = jnp.max(probs, axis=1, keepdims=True)
    i1 = jnp.min(jnp.where(probs == m1, iota8, E), axis=1, keepdims=True)
    masked = jnp.where(iota8 == i1, -1.0, probs)
    m2 = jnp.max(masked, axis=1, keepdims=True)
    i2 = jnp.min(jnp.where(masked == m2, iota8, E), axis=1, keepdims=True)
    denom = m1 + m2
    g1 = m1 / denom
    g2 = m2 / denom
    rw = (jnp.where(iota8 == i1, g1, 0.0)
          + jnp.where(iota8 == i2, g2, 0.0))
    rw_ref[...] = rw

    acc = jnp.zeros((TBLK, D), jnp.float32)
    for e in range(E):
        eh = jax.nn.gelu(jnp.dot(h, W_exp_ref[e],
                                 preferred_element_type=jnp.float32)
                         + b_exp_ref[e][None, :])
        acc = acc + rw[:, e:e + 1] * eh
    combined = acc

    raw_inv = (jnp.dot(combined, W_inv_ref[...],
                       preferred_element_type=jnp.float32)
               + b_inv_ref[...][None, :])
    q = jnp.dot(raw_inv, Wq_ref[...], preferred_element_type=jnp.float32)
    scores = lax.dot_general(q, mem_keys_ref[...],
                             (((1,), (1,)), ((), ())),
                             preferred_element_type=jnp.float32)
    scores = scores * (1.0 / jnp.sqrt(jnp.float32(MKD)))
    smax = jnp.max(scores, axis=1, keepdims=True)
    es = jnp.exp(scores - smax)
    attn = es / jnp.sum(es, axis=1, keepdims=True)
    mem_read = jnp.dot(attn, mem_vals_ref[...],
                       preferred_element_type=jnp.float32)
    mem_inv = raw_inv + mem_read
    tinv_ref[...] = (jnp.dot(mem_inv, W_head_ref[...],
                             preferred_element_type=jnp.float32)
                     + b_head_ref[...][None, :])
    out_ref[...] = (jnp.dot(combined, W_out_ref[...],
                            preferred_element_type=jnp.float32)
                    + b_out_ref[...][None, :])


def kernel(x, domain_idx, W_enc, b_enc, rotors, W_router, b_router, W_exp,
           b_exp, W_inv, b_inv, Wq, mem_keys, mem_vals, W_head, b_head,
           W_out, b_out):
    rotor = jnp.take(rotors, domain_idx, axis=0)

    n_blocks = N_TOK // TBLK
    rep = lambda *shape: pl.BlockSpec(shape, lambda i: (0,) * len(shape))
    grid_spec = pl.GridSpec(
        grid=(n_blocks,),
        in_specs=[
            pl.BlockSpec((TBLK, D), lambda i: (i, 0)),
            rep(D),
            rep(D, D),
            rep(D),
            rep(D, E),
            rep(E),
            rep(E, D, D),
            rep(E, D),
            rep(D, CD),
            rep(CD),
            rep(CD, MKD),
            rep(M, MKD),
            rep(M, CD),
            rep(CD, D),
            rep(D),
            rep(D, D),
            rep(D),
        ],
        out_specs=[
            pl.BlockSpec((TBLK, D), lambda i: (i, 0)),
            pl.BlockSpec((TBLK, E), lambda i: (i, 0)),
            pl.BlockSpec((TBLK, D), lambda i: (i, 0)),
        ],
    )
    out, rw, tinv = pl.pallas_call(
        _fused_body,
        grid_spec=grid_spec,
        out_shape=[
            jax.ShapeDtypeStruct((N_TOK, D), jnp.float32),
            jax.ShapeDtypeStruct((N_TOK, E), jnp.float32),
            jax.ShapeDtypeStruct((N_TOK, D), jnp.float32),
        ],
        compiler_params=pltpu.CompilerParams(
            dimension_semantics=("arbitrary",),
            vmem_limit_bytes=100 * 1024 * 1024,
        ),
    )(x, rotor, W_enc, b_enc, W_router, b_router, W_exp, b_exp,
      W_inv, b_inv, Wq, mem_keys, mem_vals, W_head, b_head, W_out, b_out)
    return out, rw, tinv
